# Initial kernel scaffold; baseline (speedup 1.0000x reference)
import jax
import jax.numpy as jnp
from jax.experimental import pallas as pl


def kernel(xyz, features, w0_0, b0_0, w0_1, b0_1, w1_0, b1_0, w1_1, b1_1):
    raise NotImplementedError("write your pallas kernel here")


# speedup vs baseline: 19.8894x; 19.8894x over previous
import functools

import jax
import jax.numpy as jnp
from jax import lax
from jax.experimental import pallas as pl
from jax.experimental.pallas import tpu as pltpu
from jax.experimental.pallas import tpu_sc as plsc

_B, _N, _C = 4, 8192, 16
_S = 1024
_R0, _R1 = 0.2, 0.4
_NS0, _NS1 = 16, 32
_NSTOT = _NS0 + _NS1
_D = 32

_SBLK = 256

_NW = 32
_TOT = _B * _NSTOT * _S
_PW = _TOT // _NW
_G = 128
_GRP = 4
_ROWS = _G * _GRP


def _fps_kernel(xyz_ref, out_ref):
    x0 = xyz_ref[:, 0:8, :]
    x1 = xyz_ref[:, 8:16, :]
    x2 = xyz_ref[:, 16:24, :]
    ii = (lax.broadcasted_iota(jnp.int32, (_B, 8, 1024), 1) * 1024
          + lax.broadcasted_iota(jnp.int32, (_B, 8, 1024), 2))

    def pick_coords(mask):
        mf = mask.astype(jnp.float32)

        def red(a):
            t = jnp.sum(a * mf, axis=2, keepdims=True)
            return jnp.sum(t, axis=1, keepdims=True)

        return red(x0), red(x1), red(x2)

    def row4(p0, p1, p2):
        return jnp.concatenate(
            [p0, p1, p2, jnp.zeros((_B, 1, 1), jnp.float32)], axis=2)

    p0, p1, p2 = pick_coords(ii == 0)
    out_ref[:, 0:1, :] = row4(p0, p1, p2)

    dists0 = jnp.full((_B, 8, 1024), 1e10, jnp.float32)

    def body(i, carry):
        dists, p0, p1, p2 = carry
        s0 = (x0 - p0) * (x0 - p0)
        s1 = (x1 - p1) * (x1 - p1)
        s2 = (x2 - p2) * (x2 - p2)
        d = (s0 + s2) + s1
        dists = jnp.minimum(dists, d)
        m = jnp.max(jnp.max(dists, axis=2, keepdims=True), axis=1,
                    keepdims=True)
        cand = jnp.where(dists == m, ii, _N)
        idx = jnp.min(jnp.min(cand, axis=2, keepdims=True), axis=1,
                      keepdims=True)
        q0, q1, q2 = pick_coords(ii == idx)
        out_ref[:, pl.ds(i, 1), :] = row4(q0, q1, q2)
        return (dists, q0, q1, q2)

    lax.fori_loop(1, _S, body, (dists0, p0, p1, p2))


def _bq_kernel(c_ref, x_ref, out_ref):
    cm = c_ref[0]
    xt = x_ref[0]
    inner = jnp.dot(cm, xt, preferred_element_type=jnp.float32)
    cn2 = jnp.sum(cm * cm, axis=1, keepdims=True)
    xn2 = jnp.sum(xt * xt, axis=0, keepdims=True)
    d2 = cn2 + xn2 - 2.0 * inner
    lane = lax.broadcasted_iota(jnp.int32, (_SBLK, _N), 1)
    col = 0
    for r, ns in ((_R0, _NS0), (_R1, _NS1)):
        scores = jnp.where(d2 < r * r, lane, _N)
        first = None
        for _ in range(ns):
            raw = jnp.min(scores, axis=1, keepdims=True)
            if first is None:
                first = raw
            sel = jnp.where(raw == _N, first, raw)
            sel = jnp.minimum(sel, _N - 1)
            out_ref[0, :, col:col + 1] = sel
            scores = jnp.where(scores == raw, _N, scores)
            col += 1


def _sc_gather(table, idx_flat):
    mesh = plsc.VectorSubcoreMesh(core_axis_name="c", subcore_axis_name="s")

    @functools.partial(
        pl.kernel,
        mesh=mesh,
        compiler_params=pltpu.CompilerParams(use_tc_tiling_on_sc=False),
        out_type=jax.ShapeDtypeStruct((_TOT, _D), jnp.float32),
        scratch_types=[
            pltpu.VMEM((_PW,), jnp.int32),
            pltpu.VMEM((_ROWS, _D), jnp.float32),
            pltpu.SemaphoreType.DMA,
        ],
    )
    def k(tab_hbm, idx_hbm, out_hbm, idx_v, rows_v, sem):
        wid = lax.axis_index("s") * 2 + lax.axis_index("c")
        base = wid * _PW
        pltpu.sync_copy(idx_hbm.at[pl.ds(base, _PW)], idx_v)

        def grp(g, carry):
            goff = g * _ROWS
            cps = []
            for m in range(_GRP):
                cps.append(pltpu.async_copy(
                    tab_hbm.at[idx_v.at[pl.ds(goff + m * _G, _G)]],
                    rows_v.at[pl.ds(m * _G, _G)],
                    sem))
            for cp in cps:
                cp.wait()
            pltpu.sync_copy(rows_v, out_hbm.at[pl.ds(base + goff, _ROWS)])
            return carry

        lax.fori_loop(0, _PW // _ROWS, grp, 0)

    return k(table, idx_flat)


def _mlp_kernel(ns, g_ref, c_ref, w1_ref, b1_ref, w2_ref, b2_ref, out_ref):
    c3 = c_ref[0][:, 0:3]
    sub = jnp.concatenate(
        [c3, jnp.zeros((_S, _D - 3), jnp.float32)], axis=1)
    w1 = w1_ref[...]
    b1 = b1_ref[...]
    w2 = w2_ref[...]
    b2 = b2_ref[...]
    acc = None
    for j in range(ns):
        xj = g_ref[0, j * _S:(j + 1) * _S, :] - sub
        h1 = jnp.maximum(
            jnp.dot(xj, w1, preferred_element_type=jnp.float32) + b1, 0.0)
        h2 = jnp.maximum(
            jnp.dot(h1, w2, preferred_element_type=jnp.float32) + b2, 0.0)
        acc = h2 if acc is None else jnp.maximum(acc, h2)
    out_ref[0, :, :] = acc


def kernel(xyz, features, w0_0, b0_0, w0_1, b0_1, w1_0, b1_0, w1_1, b1_1):
    xyz_t = jnp.transpose(xyz, (0, 2, 1))
    xyz24 = xyz_t.reshape(_B, 3, 8, 1024).reshape(_B, 24, 1024)

    fps_out = pl.pallas_call(
        _fps_kernel,
        grid=(1,),
        in_specs=[pl.BlockSpec((_B, 24, 1024), lambda i: (0, 0, 0))],
        out_specs=pl.BlockSpec((_B, _S, 4), lambda i: (0, 0, 0)),
        out_shape=jax.ShapeDtypeStruct((_B, _S, 4), jnp.float32),
    )(xyz24)
    new_xyz = fps_out[:, :, :3]

    xyzT4 = jnp.concatenate(
        [xyz_t, jnp.zeros((_B, 1, _N), jnp.float32)], axis=1)
    idx64 = pl.pallas_call(
        _bq_kernel,
        grid=(_B, _S // _SBLK),
        in_specs=[
            pl.BlockSpec((1, _SBLK, 4), lambda b, sb: (b, sb, 0)),
            pl.BlockSpec((1, 4, _N), lambda b, sb: (b, 0, 0)),
        ],
        out_specs=pl.BlockSpec((1, _SBLK, 64), lambda b, sb: (b, sb, 0)),
        out_shape=jax.ShapeDtypeStruct((_B, _S, 64), jnp.int32),
    )(fps_out, xyzT4)
    idx48 = idx64[:, :, :_NSTOT]

    feats_t = jnp.transpose(features, (0, 2, 1))
    table = jnp.concatenate(
        [xyz, feats_t, jnp.zeros((_B, _N, _D - 3 - _C), jnp.float32)],
        axis=2).reshape(_B * _N, _D)

    idx_slotmajor = jnp.transpose(idx48, (0, 2, 1))
    idx_flat = (idx_slotmajor
                + (jnp.arange(_B, dtype=jnp.int32) * _N)[:, None, None]
                ).reshape(_TOT)

    grouped = _sc_gather(table, idx_flat).reshape(_B, _NSTOT, _S, _D)
    g1 = grouped[:, :_NS0].reshape(_B, _NS0 * _S, _D)
    g2 = grouped[:, _NS0:].reshape(_B, _NS1 * _S, _D)

    def mlp(ns, g, w1, b1, w2, b2):
        h = w1.shape[0]
        o = w2.shape[0]
        w1p = jnp.pad(w1.T, ((0, _D - (_C + 3)), (0, 0)))
        w2p = w2.T
        return pl.pallas_call(
            functools.partial(_mlp_kernel, ns),
            grid=(_B,),
            in_specs=[
                pl.BlockSpec((1, ns * _S, _D), lambda b: (b, 0, 0)),
                pl.BlockSpec((1, _S, 4), lambda b: (b, 0, 0)),
                pl.BlockSpec((_D, h), lambda b: (0, 0)),
                pl.BlockSpec((1, h), lambda b: (0, 0)),
                pl.BlockSpec((h, o), lambda b: (0, 0)),
                pl.BlockSpec((1, o), lambda b: (0, 0)),
            ],
            out_specs=pl.BlockSpec((1, _S, o), lambda b: (b, 0, 0)),
            out_shape=jax.ShapeDtypeStruct((_B, _S, o), jnp.float32),
        )(g, fps_out, w1p, b1.reshape(1, h), w2p, b2.reshape(1, o))

    out1 = mlp(_NS0, g1, w0_0, b0_0, w0_1, b0_1)
    out2 = mlp(_NS1, g2, w1_0, b1_0, w1_1, b1_1)

    feat_out = jnp.transpose(
        jnp.concatenate([out1, out2], axis=2), (0, 2, 1))
    return new_xyz, feat_out
